# Initial kernel scaffold; baseline (speedup 1.0000x reference)
import jax
import jax.numpy as jnp
from jax.experimental import pallas as pl


def kernel(x, edge_index, edge_attr, frag_batch, graph_batch, en_W, en_b, mlp_W1, mlp_b1, mlp_W2, mlp_b2, bn_g, bn_b):
    raise NotImplementedError("write your pallas kernel here")


# speedup vs baseline: 3.0559x; 3.0559x over previous
import functools

import jax
import jax.numpy as jnp
from jax import lax
from jax.experimental import pallas as pl
from jax.experimental.pallas import tpu as pltpu
from jax.experimental.pallas import tpu_sc as plsc

N = 10000
E = 320000
D = 128
L = 3
NF = 512
NG = 64

NC = 2
NS = 16
NW = NC * NS
C = 128
NCHUNK = E // C
TBASE = NCHUNK // NW
TEXTRA = NCHUNK - TBASE * NW
ROWS_A = 624
ROWS_LAST = N - 15 * ROWS_A


BE = 1280


def _k1_body(ea_ref, w_ref, b_ref, o_ref):
    o_ref[...] = lax.dot_general(
        ea_ref[...], w_ref[...], (((1,), (1,)), ((), ())),
        preferred_element_type=jnp.float32) + b_ref[...]


def _k1(ea, w, b):
    grid = (E // BE,)
    blk = pl.BlockSpec((BE, D), lambda i: (i, 0))
    return pl.pallas_call(
        _k1_body,
        grid=grid,
        in_specs=[
            blk,
            pl.BlockSpec((D, D), lambda i: (0, 0)),
            pl.BlockSpec((1, D), lambda i: (0, 0)),
        ],
        out_specs=blk,
        out_shape=jax.ShapeDtypeStruct((E, D), jnp.float32),
    )(ea, w, b)


def _k2_body(h_hbm, ea_hbm, src_hbm, dst_hbm, zero_hbm, out_hbm,
             src_v, dst_v, ea_v, hr_v, sem_h, agg_sh):
    cid = lax.axis_index("c")
    sid = lax.axis_index("s")
    wid = cid * NS + sid
    start = wid * TBASE + jnp.minimum(wid, TEXTRA)
    nchunks = TBASE + jnp.where(wid < TEXTRA, 1, 0)

    row0 = pl.multiple_of(sid * ROWS_A, 8)

    @pl.when(sid < NS - 1)
    def _zero_main():
        pltpu.sync_copy(zero_hbm.at[pl.ds(row0, ROWS_A)],
                        agg_sh.at[pl.ds(row0, ROWS_A)])

    @pl.when(sid == NS - 1)
    def _zero_last():
        pltpu.sync_copy(zero_hbm.at[pl.ds((NS - 1) * ROWS_A, ROWS_LAST)],
                        agg_sh.at[pl.ds((NS - 1) * ROWS_A, ROWS_LAST)])

    plsc.subcore_barrier()

    def chunk_body(j, carry):
        @pl.when(j < nchunks)
        def _do_chunk():
            g = start + j
            off = pl.multiple_of(g * C, 8)
            pltpu.sync_copy(src_hbm.at[g], src_v)
            pltpu.sync_copy(dst_hbm.at[g], dst_v)
            pltpu.sync_copy(ea_hbm.at[pl.ds(off, C)], ea_v)
            pltpu.async_copy(h_hbm.at[src_v], hr_v, sem_h).wait()

            def row_body(i, carry2):
                for k in range(D // 16):
                    sl = (i, pl.ds(k * 16, 16))
                    ea_v[sl] = jnp.maximum(ea_v[sl] + hr_v[sl], 0.0)
                return carry2

            lax.fori_loop(0, C, row_body, 0)
            pltpu.sync_copy(ea_v, agg_sh.at[dst_v], add=True)

        return carry

    lax.fori_loop(0, TBASE + 1, chunk_body, 0)
    plsc.subcore_barrier()

    @pl.when(sid < NS - 1)
    def _dump_main():
        pltpu.sync_copy(agg_sh.at[pl.ds(row0, ROWS_A)],
                        out_hbm.at[cid, pl.ds(row0, ROWS_A)])

    @pl.when(sid == NS - 1)
    def _dump_last():
        pltpu.sync_copy(agg_sh.at[pl.ds((NS - 1) * ROWS_A, ROWS_LAST)],
                        out_hbm.at[cid, pl.ds((NS - 1) * ROWS_A, ROWS_LAST)])


@functools.partial(jax.jit, static_argnames=())
def _k2(h, ea, src, dst, zeros):
    mesh = plsc.VectorSubcoreMesh(core_axis_name="c", subcore_axis_name="s",
                                  num_cores=NC, num_subcores=NS)
    f = pl.kernel(
        _k2_body,
        out_type=jax.ShapeDtypeStruct((NC, N, D), jnp.float32),
        mesh=mesh,
        scratch_types=[
            pltpu.VMEM((C,), jnp.int32),
            pltpu.VMEM((C,), jnp.int32),
            pltpu.VMEM((C, D), jnp.float32),
            pltpu.VMEM((C, D), jnp.float32),
            pltpu.SemaphoreType.DMA,
            pltpu.VMEM_SHARED((N, D), jnp.float32),
        ],
    )
    return f(h, ea, src, dst, zeros)


def _k3_body(h_ref, agg_ref, w1_ref, b1_ref, w2_ref, b2_ref, g_ref, bb_ref,
             o_ref):
    z = h_ref[...] + agg_ref[0] + agg_ref[1]
    z1 = lax.dot_general(z, w1_ref[...], (((1,), (1,)), ((), ())),
                         preferred_element_type=jnp.float32) + b1_ref[...]
    z1 = jnp.maximum(z1, 0.0)
    z2 = lax.dot_general(z1, w2_ref[...], (((1,), (1,)), ((), ())),
                         preferred_element_type=jnp.float32) + b2_ref[...]
    z3 = jnp.maximum(z2, 0.0)
    mu = jnp.mean(z3, axis=0, keepdims=True)
    dzm = z3 - mu
    var = jnp.mean(dzm * dzm, axis=0, keepdims=True)
    o_ref[...] = dzm / jnp.sqrt(var + 1e-5) * g_ref[...] + bb_ref[...]


def _k3(h, agg, w1, b1, w2, b2, g, bb):
    return pl.pallas_call(
        _k3_body,
        out_shape=jax.ShapeDtypeStruct((N, D), jnp.float32),
    )(h, agg, w1, b1, w2, b2, g, bb)


NB = 1000


def _k4_body(h_ref, fb_ref, gb_ref, of_ref, og_ref, cf_ref, cg_ref):
    i = pl.program_id(0)
    nsteps = pl.num_programs(0)
    h = h_ref[...]
    ones = jnp.ones((NB, 1), dtype=jnp.float32)

    @pl.when(i == 0)
    def _init():
        of_ref[...] = jnp.zeros_like(of_ref)
        og_ref[...] = jnp.zeros_like(og_ref)
        cf_ref[...] = jnp.zeros_like(cf_ref)
        cg_ref[...] = jnp.zeros_like(cg_ref)

    for (b_ref, o_ref, c_ref, nseg) in (
            (fb_ref, of_ref, cf_ref, NF), (gb_ref, og_ref, cg_ref, NG)):
        seg_ids = lax.broadcasted_iota(
            jnp.int32, (1, nseg), 1).astype(jnp.float32)
        onehot = (b_ref[...] == seg_ids).astype(jnp.float32)
        o_ref[...] += lax.dot_general(
            onehot, h, (((0,), (0,)), ((), ())),
            preferred_element_type=jnp.float32,
        precision=lax.Precision.HIGHEST)
        c_ref[...] += lax.dot_general(
            onehot, ones, (((0,), (0,)), ((), ())),
            preferred_element_type=jnp.float32,
        precision=lax.Precision.HIGHEST)

    @pl.when(i == nsteps - 1)
    def _finish():
        of_ref[...] /= jnp.sqrt(jnp.maximum(cf_ref[...], 1.0))
        og_ref[...] /= jnp.sqrt(jnp.maximum(cg_ref[...], 1.0))


def _k4(h, fbf, gbf):
    grid = (N // NB,)
    return pl.pallas_call(
        _k4_body,
        grid=grid,
        in_specs=[
            pl.BlockSpec((NB, D), lambda i: (i, 0)),
            pl.BlockSpec((NB, 1), lambda i: (i, 0)),
            pl.BlockSpec((NB, 1), lambda i: (i, 0)),
        ],
        out_specs=(
            pl.BlockSpec((NF, D), lambda i: (0, 0)),
            pl.BlockSpec((NG, D), lambda i: (0, 0)),
        ),
        out_shape=(
            jax.ShapeDtypeStruct((NF, D), jnp.float32),
            jax.ShapeDtypeStruct((NG, D), jnp.float32),
        ),
        scratch_shapes=[
            pltpu.VMEM((NF, 1), jnp.float32),
            pltpu.VMEM((NG, 1), jnp.float32),
        ],
    )(h, fbf, gbf)


def kernel(x, edge_index, edge_attr, frag_batch, graph_batch,
           en_W, en_b, mlp_W1, mlp_b1, mlp_W2, mlp_b2, bn_g, bn_b):
    src = edge_index[0].reshape(NCHUNK, C)
    dst = edge_index[1].reshape(NCHUNK, C)
    zeros = jnp.zeros((N, D), jnp.float32)

    h = x
    ea = edge_attr
    for i in range(L):
        ea = _k1(ea, en_W[i], en_b[i][None, :])
        agg = _k2(h, ea, src, dst, zeros)
        h = _k3(h, agg, mlp_W1[i], mlp_b1[i][None, :], mlp_W2[i],
                mlp_b2[i][None, :], bn_g[i][None, :], bn_b[i][None, :])

    fbf = frag_batch.astype(jnp.float32)[:, None]
    gbf = graph_batch.astype(jnp.float32)[:, None]
    return _k4(h, fbf, gbf)
